# Initial kernel scaffold; baseline (speedup 1.0000x reference)
import jax
import jax.numpy as jnp
from jax.experimental import pallas as pl


def kernel(x, edge_index, W, att_src, att_dst, bias):
    raise NotImplementedError("write your pallas kernel here")


# speedup vs baseline: 20.9354x; 20.9354x over previous
import dataclasses

import jax
import jax.numpy as jnp
from jax import lax
from jax.experimental import pallas as pl
from jax.experimental.pallas import tpu as pltpu
from jax.experimental.pallas import tpu_sc as plsc

_N = 10000
_E = 320000
_D = 128
_NC = 2
_NS = 16
_NW = _NC * _NS
_EPW = _E // _NW
_CH = 80
_NCH = _EPW // _CH
_NP = 10240
_RPS = _NP // _NS
_DR = _NP // 16
_DRS = _DR // _NS
_RPW = _NP // _NW
_CB = 80
_NCB = _RPW // _CB
_DPV = 24
_L = 16


def _tc1_body(x_ref, w_ref, as_ref, ad_ref, h_ref, asrc_ref, adst_ref,
              ss_ref, c_ref):
    h = lax.dot_general(x_ref[...], w_ref[...], (((1,), (0,)), ((), ())),
                        preferred_element_type=jnp.float32)
    h_ref[0:_N, :] = h
    h_ref[_N:_NP, :] = jnp.zeros((_NP - _N, _D), jnp.float32)
    a_s = lax.dot_general(h, as_ref[...], (((1,), (0,)), ((), ())),
                          preferred_element_type=jnp.float32)
    a_d = lax.dot_general(h, ad_ref[...], (((1,), (0,)), ((), ())),
                          preferred_element_type=jnp.float32)
    asrc_ref[0:_N, :] = a_s
    asrc_ref[_N:_NP, :] = jnp.zeros((_NP - _N, 1), jnp.float32)
    adst_ref[0:_N, :] = a_d
    adst_ref[_N:_NP, :] = jnp.zeros((_NP - _N, 1), jnp.float32)
    t = jnp.max(a_s) + jnp.max(a_d)
    c = jnp.maximum(t, 0.2 * t)
    u = a_s + a_d
    ss_ref[0:_N, :] = jnp.exp(jnp.maximum(u, 0.2 * u) - c)
    ss_ref[_N:_NP, :] = jnp.ones((_NP - _N, 1), jnp.float32)
    c_ref[...] = jnp.broadcast_to(c, (1, 1))


_tc1 = pl.pallas_call(
    _tc1_body,
    out_shape=[
        jax.ShapeDtypeStruct((_NP, _D), jnp.float32),
        jax.ShapeDtypeStruct((_NP, 1), jnp.float32),
        jax.ShapeDtypeStruct((_NP, 1), jnp.float32),
        jax.ShapeDtypeStruct((_NP, 1), jnp.float32),
        jax.ShapeDtypeStruct((1, 1), jnp.float32),
    ],
)


def _sc1_body(h_hbm, src_hbm, dst_hbm, asrc_hbm, adst_hbm, cvec_hbm, zrows_hbm,
              parts_hbm, denp_hbm,
              asrc_v, adst_v, c_v, srci_v, dsti_v, dstr_v, s_v, srow_v, rows_v,
              acc_sh, den_sh, sem):
    cid = lax.axis_index("c")
    sid = lax.axis_index("s")
    wid = cid * _NS + sid

    pltpu.sync_copy(zrows_hbm, acc_sh.at[pl.ds(sid * _RPS, _RPS)])
    pltpu.sync_copy(zrows_hbm.at[pl.ds(0, _DRS)],
                    den_sh.at[pl.ds(sid * _DRS, _DRS)])
    pltpu.sync_copy(zrows_hbm.at[pl.ds(0, _CH)], srow_v)
    pltpu.sync_copy(asrc_hbm, asrc_v)
    pltpu.sync_copy(adst_hbm, adst_v)
    pltpu.sync_copy(cvec_hbm, c_v)
    plsc.subcore_barrier()

    creg = c_v[...]
    base0 = wid * _EPW

    @pl.loop(0, _NCH)
    def _(ch):
        base = base0 + ch * _CH
        pltpu.sync_copy(src_hbm.at[pl.ds(base, _CH)], srci_v)
        pltpu.sync_copy(dst_hbm.at[pl.ds(base, _CH)], dsti_v.at[0])
        cp = pltpu.async_copy(h_hbm.at[srci_v], rows_v, sem)
        for g in range(_CH // _L):
            sv = srci_v[pl.ds(g * _L, _L)]
            dv = dsti_v[0, pl.ds(g * _L, _L)]
            u = plsc.load_gather(asrc_v, [sv]) + plsc.load_gather(adst_v, [dv])
            e = jnp.maximum(u, 0.2 * u)
            s = jnp.exp(e - creg)
            s_v[pl.ds(g * _L, _L)] = s
            rows16 = g * _L + lax.iota(jnp.int32, _L)
            plsc.store_scatter(srow_v, [rows16, dv & 15], s)
            dstr_v[0, pl.ds(g * _L, _L)] = dv >> 4
        cp.wait()

        @pl.loop(0, _CH)
        def _(r):
            sp = plsc.load_gather(s_v, [jnp.full((_L,), r, jnp.int32)])
            for j in range(_D // _L):
                sl = (r, pl.ds(j * _L, _L))
                rows_v[sl] = rows_v[sl] * sp

        pltpu.sync_copy(rows_v, acc_sh.at[dsti_v.at[0]], add=True)
        pltpu.sync_copy(srow_v, den_sh.at[dstr_v.at[0]], add=True)
        for g in range(_CH // _L):
            dv = dsti_v[0, pl.ds(g * _L, _L)]
            rows16 = g * _L + lax.iota(jnp.int32, _L)
            plsc.store_scatter(srow_v, [rows16, dv & 15],
                               jnp.zeros((_L,), jnp.float32))

    plsc.subcore_barrier()
    pltpu.sync_copy(acc_sh.at[pl.ds(sid * _RPS, _RPS)],
                    parts_hbm.at[cid].at[pl.ds(sid * _RPS, _RPS)])
    pltpu.sync_copy(den_sh.at[pl.ds(sid * _DRS, _DRS)],
                    denp_hbm.at[cid].at[pl.ds(sid * _DRS, _DRS)])


_sc_params = pltpu.CompilerParams()
if "needs_layout_passes" in pltpu.CompilerParams.__dataclass_fields__:
    _sc_params = dataclasses.replace(_sc_params, needs_layout_passes=False)

_sc1 = pl.kernel(
    _sc1_body,
    out_type=[
        jax.ShapeDtypeStruct((_NC, _NP, _D), jnp.float32),
        jax.ShapeDtypeStruct((_NC, _DR, _D), jnp.float32),
    ],
    mesh=plsc.VectorSubcoreMesh(core_axis_name="c", subcore_axis_name="s"),
    compiler_params=_sc_params,
    scratch_types=[
        pltpu.VMEM((_NP,), jnp.float32),
        pltpu.VMEM((_NP,), jnp.float32),
        pltpu.VMEM((_L,), jnp.float32),
        pltpu.VMEM((_CH,), jnp.int32),
        pltpu.VMEM((1, _CH), jnp.int32),
        pltpu.VMEM((1, _CH), jnp.int32),
        pltpu.VMEM((_CH,), jnp.float32),
        pltpu.VMEM((_CH, _D), jnp.float32),
        pltpu.VMEM((_CH, _D), jnp.float32),
        pltpu.VMEM_SHARED((_NP, _D), jnp.float32),
        pltpu.VMEM_SHARED((_DR, _D), jnp.float32),
        pltpu.SemaphoreType.DMA,
    ],
)


def _sc2_body(parts_hbm, denp_hbm, h_hbm, ss_hbm, outd_hbm,
              p0_v, p1_v, h_v, ss_v, d0_v, d1_v, sem):
    cid = lax.axis_index("c")
    sid = lax.axis_index("s")
    wid = cid * _NS + sid
    basew = wid * _RPW

    off = (wid & 1) * 4
    dbase = pl.multiple_of(wid * (_RPW // 16) - off, 8)
    pltpu.sync_copy(denp_hbm.at[0].at[pl.ds(dbase, _DPV)], d0_v)
    pltpu.sync_copy(denp_hbm.at[1].at[pl.ds(dbase, _DPV)], d1_v)

    for cc in range(_NCB):
        base = basew + cc * _CB
        pltpu.sync_copy(parts_hbm.at[0].at[pl.ds(base, _CB)], p0_v)
        pltpu.sync_copy(parts_hbm.at[1].at[pl.ds(base, _CB)], p1_v)
        pltpu.sync_copy(h_hbm.at[pl.ds(base, _CB)], h_v)
        pltpu.sync_copy(ss_hbm.at[pl.ds(base, _CB)], ss_v)

        @pl.loop(0, _CB)
        def _(r):
            arow = jnp.full((_L,), off + cc * (_CB // 16) + (r >> 4),
                            jnp.int32)
            acol = jnp.full((_L,), r & 15, jnp.int32)
            den = (plsc.load_gather(d0_v, [arow, acol]) +
                   plsc.load_gather(d1_v, [arow, acol]))
            ssr = plsc.load_gather(ss_v, [jnp.full((_L,), r, jnp.int32)])
            rec = 1.0 / (den + ssr + 1e-16)
            for j in range(_D // _L):
                sl = (r, pl.ds(j * _L, _L))
                p0_v[sl] = (p0_v[sl] + p1_v[sl] + ssr * h_v[sl]) * rec

        pltpu.sync_copy(p0_v, outd_hbm.at[pl.ds(base, _CB)])


_sc2 = pl.kernel(
    _sc2_body,
    out_type=jax.ShapeDtypeStruct((_NP, _D), jnp.float32),
    mesh=plsc.VectorSubcoreMesh(core_axis_name="c", subcore_axis_name="s"),
    compiler_params=_sc_params,
    scratch_types=[
        pltpu.VMEM((_CB, _D), jnp.float32),
        pltpu.VMEM((_CB, _D), jnp.float32),
        pltpu.VMEM((_CB, _D), jnp.float32),
        pltpu.VMEM((_CB,), jnp.float32),
        pltpu.VMEM((_DPV, _D), jnp.float32),
        pltpu.VMEM((_DPV, _D), jnp.float32),
        pltpu.SemaphoreType.DMA,
    ],
)


def _tc2_body(outd_ref, b_ref, o_ref):
    o_ref[...] = jnp.tanh(outd_ref[0:_N, :] + b_ref[...])


_tc2 = pl.pallas_call(
    _tc2_body,
    out_shape=jax.ShapeDtypeStruct((_N, _D), jnp.float32),
)


@jax.jit
def kernel(x, edge_index, W, att_src, att_dst, bias):
    src = edge_index[0]
    dst = edge_index[1]
    hp, asrc, adst, ssp, c = _tc1(x, W, att_src.reshape(_D, 1),
                                  att_dst.reshape(_D, 1))
    cvec = jnp.broadcast_to(c.reshape(1), (_L,))
    zrows = jnp.zeros((_RPS, _D), jnp.float32)
    parts, denp = _sc1(hp, src, dst, asrc.reshape(_NP), adst.reshape(_NP),
                       cvec, zrows)
    outd = _sc2(parts, denp, hp, ssp.reshape(_NP))
    return _tc2(outd, bias.reshape(1, _D))


# speedup vs baseline: 23.3730x; 1.1164x over previous
import dataclasses

import jax
import jax.numpy as jnp
from jax import lax
from jax.experimental import pallas as pl
from jax.experimental.pallas import tpu as pltpu
from jax.experimental.pallas import tpu_sc as plsc

_N = 10000
_E = 320000
_D = 128
_NC = 2
_NS = 16
_NW = _NC * _NS
_CH = 64
_NCHM = 156
_REM_BASE = _NCHM * _CH * _NW
_EPW = _NCHM * _CH
_NP = 10240
_RPS = _NP // _NS
_DRW = 80
_NB = _NP // 128
_L = 16


def _tc1_body(x_ref, w_ref, as_ref, ad_ref, h_ref, asrc_ref, adst_ref,
              ss_ref, c_ref):
    h = lax.dot_general(x_ref[...], w_ref[...], (((1,), (0,)), ((), ())),
                        preferred_element_type=jnp.float32)
    h_ref[0:_N, :] = h
    h_ref[_N:_NP, :] = jnp.zeros((_NP - _N, _D), jnp.float32)
    a_s = lax.dot_general(h, as_ref[...], (((1,), (0,)), ((), ())),
                          preferred_element_type=jnp.float32)
    a_d = lax.dot_general(h, ad_ref[...], (((1,), (0,)), ((), ())),
                          preferred_element_type=jnp.float32)
    asrc_ref[0:_N, :] = a_s
    asrc_ref[_N:_NP, :] = jnp.zeros((_NP - _N, 1), jnp.float32)
    adst_ref[0:_N, :] = a_d
    adst_ref[_N:_NP, :] = jnp.zeros((_NP - _N, 1), jnp.float32)
    t = jnp.max(a_s) + jnp.max(a_d)
    c = jnp.maximum(t, 0.2 * t)
    u = a_s + a_d
    ss_ref[0:_N, :] = jnp.exp(jnp.maximum(u, 0.2 * u) - c)
    ss_ref[_N:_NP, :] = jnp.ones((_NP - _N, 1), jnp.float32)
    c_ref[...] = jnp.broadcast_to(c, (1, 1))


_tc1 = pl.pallas_call(
    _tc1_body,
    out_shape=[
        jax.ShapeDtypeStruct((_NP, _D), jnp.float32),
        jax.ShapeDtypeStruct((_NP, 1), jnp.float32),
        jax.ShapeDtypeStruct((_NP, 1), jnp.float32),
        jax.ShapeDtypeStruct((_NP, 1), jnp.float32),
        jax.ShapeDtypeStruct((1, 1), jnp.float32),
    ],
)


def _sc1_body(h_hbm, src_hbm, dst_hbm, asrc_hbm, adst_hbm, cvec_hbm, zrows_hbm,
              parts_hbm, denp_hbm,
              asrc_v, adst_v, c_v, srci0, srci1, dsti0, dsti1, s0, s1,
              rows0, rows1, den_v, idn_v, acc_sh, den_sh,
              semg0, semg1, sems0, sems1):
    cid = lax.axis_index("c")
    sid = lax.axis_index("s")
    wid = cid * _NS + sid

    pltpu.sync_copy(zrows_hbm, acc_sh.at[pl.ds(sid * _RPS, _RPS)])
    pltpu.sync_copy(zrows_hbm.at[pl.ds(0, _DRW // _NS)],
                    den_sh.at[pl.ds(sid * (_DRW // _NS), _DRW // _NS)])
    pltpu.sync_copy(zrows_hbm.at[pl.ds(0, _DRW)], den_v)
    pltpu.sync_copy(asrc_hbm, asrc_v)
    pltpu.sync_copy(adst_hbm, adst_v)
    pltpu.sync_copy(cvec_hbm, c_v)
    for g in range(_DRW // _L):
        idn_v[0, pl.ds(g * _L, _L)] = g * _L + lax.iota(jnp.int32, _L)
    plsc.subcore_barrier()

    creg = c_v[...]
    basew = wid * _EPW
    bufs = ((srci0, dsti0, s0, rows0, semg0, sems0),
            (srci1, dsti1, s1, rows1, semg1, sems1))

    def fetch(base, srci, dsti, rows, semg):
        pltpu.sync_copy(src_hbm.at[pl.ds(base, _CH)], srci)
        pltpu.sync_copy(dst_hbm.at[pl.ds(base, _CH)], dsti.at[0])
        cp = pltpu.async_copy(h_hbm.at[srci], rows, semg)
        return cp

    def weights(srci, dsti, s_v):
        for g in range(_CH // _L):
            sv = srci[pl.ds(g * _L, _L)]
            dv = dsti[0, pl.ds(g * _L, _L)]
            u = plsc.load_gather(asrc_v, [sv]) + plsc.load_gather(adst_v, [dv])
            e = jnp.maximum(u, 0.2 * u)
            s = jnp.exp(e - creg)
            s_v[pl.ds(g * _L, _L)] = s
            plsc.addupdate_scatter(den_v, [dv >> 7, dv & 127], s)

    def scale(rows, s_v):
        @pl.loop(0, _CH)
        def _(r):
            sp = plsc.load_gather(s_v, [jnp.full((_L,), r, jnp.int32)])
            for j in range(_D // _L):
                sl = (r, pl.ds(j * _L, _L))
                rows[sl] = rows[sl] * sp

    fetch(basew, srci0, dsti0, rows0, semg0)
    weights(srci0, dsti0, s0)

    @pl.loop(0, _NCHM // 2)
    def _(i):
        for b in range(2):
            k = i * 2 + b
            srci, dsti, s_v, rows, semg, sems = bufs[b]
            nsrci, ndsti, ns_v, nrows, nsemg, nsems = bufs[1 - b]
            pltpu.make_async_copy(h_hbm.at[srci], rows, semg).wait()
            scale(rows, s_v)
            pltpu.async_copy(rows, acc_sh.at[dsti.at[0]], sems, add=True)

            @pl.when(k + 1 < _NCHM)
            def _():
                @pl.when(k >= 1)
                def _():
                    pltpu.make_async_copy(
                        nrows, acc_sh.at[ndsti.at[0]], nsems).wait()
                fetch(basew + (k + 1) * _CH, nsrci, ndsti, nrows, nsemg)
                weights(nsrci, ndsti, ns_v)

    pltpu.make_async_copy(rows0, acc_sh.at[dsti0.at[0]], sems0).wait()
    pltpu.make_async_copy(rows1, acc_sh.at[dsti1.at[0]], sems1).wait()

    @pl.when(wid < 8)
    def _():
        cp = fetch(_REM_BASE + wid * _CH, srci0, dsti0, rows0, semg0)
        weights(srci0, dsti0, s0)
        cp.wait()
        scale(rows0, s0)
        pltpu.sync_copy(rows0, acc_sh.at[dsti0.at[0]], add=True)

    pltpu.sync_copy(den_v, den_sh.at[idn_v.at[0]], add=True)
    plsc.subcore_barrier()

    pltpu.sync_copy(acc_sh.at[pl.ds(sid * _RPS, _RPS)],
                    parts_hbm.at[cid].at[pl.ds(sid * _RPS, _RPS)])

    @pl.when(sid == 0)
    def _():
        pltpu.sync_copy(den_sh, denp_hbm.at[cid])


_sc_params = pltpu.CompilerParams()
if "needs_layout_passes" in pltpu.CompilerParams.__dataclass_fields__:
    _sc_params = dataclasses.replace(_sc_params, needs_layout_passes=False)

_sc1 = pl.kernel(
    _sc1_body,
    out_type=[
        jax.ShapeDtypeStruct((_NC, _NP, _D), jnp.float32),
        jax.ShapeDtypeStruct((_NC, _DRW, _D), jnp.float32),
    ],
    mesh=plsc.VectorSubcoreMesh(core_axis_name="c", subcore_axis_name="s"),
    compiler_params=_sc_params,
    scratch_types=[
        pltpu.VMEM((_NP,), jnp.float32),
        pltpu.VMEM((_NP,), jnp.float32),
        pltpu.VMEM((_L,), jnp.float32),
        pltpu.VMEM((_CH,), jnp.int32),
        pltpu.VMEM((_CH,), jnp.int32),
        pltpu.VMEM((1, _CH), jnp.int32),
        pltpu.VMEM((1, _CH), jnp.int32),
        pltpu.VMEM((_CH,), jnp.float32),
        pltpu.VMEM((_CH,), jnp.float32),
        pltpu.VMEM((_CH, _D), jnp.float32),
        pltpu.VMEM((_CH, _D), jnp.float32),
        pltpu.VMEM((_DRW, _D), jnp.float32),
        pltpu.VMEM((1, _DRW), jnp.int32),
        pltpu.VMEM_SHARED((_NP, _D), jnp.float32),
        pltpu.VMEM_SHARED((_DRW, _D), jnp.float32),
        pltpu.SemaphoreType.DMA,
        pltpu.SemaphoreType.DMA,
        pltpu.SemaphoreType.DMA,
        pltpu.SemaphoreType.DMA,
    ],
)


def _sc2_body(parts_hbm, denp_hbm, h_hbm, ss_hbm, outd_hbm,
              p0_v, p1_v, h_v, ss_v, d0_v, d1_v, sem):
    cid = lax.axis_index("c")
    sid = lax.axis_index("s")
    wid = cid * _NS + sid

    pltpu.sync_copy(denp_hbm.at[0], d0_v)
    pltpu.sync_copy(denp_hbm.at[1], d1_v)

    nblk = jnp.where(wid < _NS, 3, 2)
    blk0 = jnp.where(wid < _NS, 3 * wid, 2 * wid + _NS)

    for q in range(3):
        @pl.when(q < nblk)
        def _():
            j = blk0 + q
            base = j * _D
            pltpu.sync_copy(parts_hbm.at[0].at[pl.ds(base, _D)], p0_v)
            pltpu.sync_copy(parts_hbm.at[1].at[pl.ds(base, _D)], p1_v)
            pltpu.sync_copy(h_hbm.at[pl.ds(base, _D)], h_v)
            pltpu.sync_copy(ss_hbm.at[pl.ds(base, _D)], ss_v)

            @pl.loop(0, _D)
            def _(r):
                jrow = jnp.full((_L,), j, jnp.int32)
                rcol = jnp.full((_L,), r, jnp.int32)
                den = (plsc.load_gather(d0_v, [jrow, rcol]) +
                       plsc.load_gather(d1_v, [jrow, rcol]))
                ssr = plsc.load_gather(ss_v, [rcol])
                rec = 1.0 / (den + ssr + 1e-16)
                for jj in range(_D // _L):
                    sl = (r, pl.ds(jj * _L, _L))
                    p0_v[sl] = (p0_v[sl] + p1_v[sl] + ssr * h_v[sl]) * rec

            pltpu.sync_copy(p0_v, outd_hbm.at[pl.ds(base, _D)])


_sc2 = pl.kernel(
    _sc2_body,
    out_type=jax.ShapeDtypeStruct((_NP, _D), jnp.float32),
    mesh=plsc.VectorSubcoreMesh(core_axis_name="c", subcore_axis_name="s"),
    compiler_params=_sc_params,
    scratch_types=[
        pltpu.VMEM((_D, _D), jnp.float32),
        pltpu.VMEM((_D, _D), jnp.float32),
        pltpu.VMEM((_D, _D), jnp.float32),
        pltpu.VMEM((_D,), jnp.float32),
        pltpu.VMEM((_DRW, _D), jnp.float32),
        pltpu.VMEM((_DRW, _D), jnp.float32),
        pltpu.SemaphoreType.DMA,
    ],
)


def _tc2_body(outd_ref, b_ref, o_ref):
    o_ref[...] = jnp.tanh(outd_ref[0:_N, :] + b_ref[...])


_tc2 = pl.pallas_call(
    _tc2_body,
    out_shape=jax.ShapeDtypeStruct((_N, _D), jnp.float32),
)


@jax.jit
def kernel(x, edge_index, W, att_src, att_dst, bias):
    src = edge_index[0]
    dst = edge_index[1]
    hp, asrc, adst, ssp, c = _tc1(x, W, att_src.reshape(_D, 1),
                                  att_dst.reshape(_D, 1))
    cvec = jnp.broadcast_to(c.reshape(1), (_L,))
    zrows = jnp.zeros((_RPS, _D), jnp.float32)
    parts, denp = _sc1(hp, src, dst, asrc.reshape(_NP), adst.reshape(_NP),
                       cvec, zrows)
    outd = _sc2(parts, denp, hp, ssp.reshape(_NP))
    return _tc2(outd, bias.reshape(1, _D))


# speedup vs baseline: 23.8246x; 1.0193x over previous
import dataclasses

import jax
import jax.numpy as jnp
from jax import lax
from jax.experimental import pallas as pl
from jax.experimental.pallas import tpu as pltpu
from jax.experimental.pallas import tpu_sc as plsc

_N = 10000
_E = 320000
_D = 128
_NC = 2
_NS = 16
_NW = _NC * _NS
_CH = 32
_NCHM = 312
_REM_BASE = _NCHM * _CH * _NW
_EPW = _NCHM * _CH
_NP = 10112
_RPS = _NP // _NS
_AT = 10000
_DRW = 80
_NB = _NP // _D
_L = 16


def _tc1_body(x_ref, w_ref, as_ref, ad_ref, h_ref, asrc_ref, adst_ref,
              ss_ref, c_ref):
    h = lax.dot_general(x_ref[...], w_ref[...], (((1,), (0,)), ((), ())),
                        preferred_element_type=jnp.float32)
    h_ref[0:_N, :] = h
    h_ref[_N:_NP, :] = jnp.zeros((_NP - _N, _D), jnp.float32)
    a_s = lax.dot_general(h, as_ref[...], (((1,), (0,)), ((), ())),
                          preferred_element_type=jnp.float32)
    a_d = lax.dot_general(h, ad_ref[...], (((1,), (0,)), ((), ())),
                          preferred_element_type=jnp.float32)
    asrc_ref[0:_N, :] = a_s
    asrc_ref[_N:_NP, :] = jnp.zeros((_NP - _N, 1), jnp.float32)
    adst_ref[0:_N, :] = a_d
    adst_ref[_N:_NP, :] = jnp.zeros((_NP - _N, 1), jnp.float32)
    t = jnp.max(a_s) + jnp.max(a_d)
    c = jnp.maximum(t, 0.2 * t)
    u = a_s + a_d
    ss_ref[0:_N, :] = jnp.exp(jnp.maximum(u, 0.2 * u) - c)
    ss_ref[_N:_NP, :] = jnp.ones((_NP - _N, 1), jnp.float32)
    c_ref[...] = jnp.broadcast_to(c, (1, 1))


_tc1 = pl.pallas_call(
    _tc1_body,
    out_shape=[
        jax.ShapeDtypeStruct((_NP, _D), jnp.float32),
        jax.ShapeDtypeStruct((_NP, 1), jnp.float32),
        jax.ShapeDtypeStruct((_NP, 1), jnp.float32),
        jax.ShapeDtypeStruct((_NP, 1), jnp.float32),
        jax.ShapeDtypeStruct((1, 1), jnp.float32),
    ],
)


def _sc1_body(h_hbm, src_hbm, dst_hbm, asrc_hbm, adst_hbm, cvec_hbm, zrows_hbm,
              parts_hbm, denp_hbm,
              asrc_v, adst_v, c_v, srci0, srci1, srci2, dsti0, dsti1, dsti2,
              s0, s1, s2, rows0, rows1, rows2, den_v, idn_v, acc_sh, den_sh,
              semg0, semg1, semg2, sems0, sems1, sems2):
    cid = lax.axis_index("c")
    sid = lax.axis_index("s")
    wid = cid * _NS + sid

    pltpu.sync_copy(zrows_hbm, acc_sh.at[pl.ds(sid * _RPS, _RPS)])
    pltpu.sync_copy(zrows_hbm.at[pl.ds(0, _DRW // _NS)],
                    den_sh.at[pl.ds(sid * (_DRW // _NS), _DRW // _NS)])
    pltpu.sync_copy(zrows_hbm.at[pl.ds(0, _DRW)], den_v)
    pltpu.sync_copy(asrc_hbm.at[pl.ds(0, _AT)], asrc_v)
    pltpu.sync_copy(adst_hbm.at[pl.ds(0, _AT)], adst_v)
    pltpu.sync_copy(cvec_hbm, c_v)
    for g in range(_DRW // _L):
        idn_v[0, pl.ds(g * _L, _L)] = g * _L + lax.iota(jnp.int32, _L)
    plsc.subcore_barrier()

    creg = c_v[...]
    basew = wid * _EPW
    bufs = ((srci0, dsti0, s0, rows0, semg0, sems0),
            (srci1, dsti1, s1, rows1, semg1, sems1),
            (srci2, dsti2, s2, rows2, semg2, sems2))

    def fetch(base, buf):
        srci, dsti, _, rows, semg, _ = buf
        pltpu.sync_copy(src_hbm.at[pl.ds(base, _CH)], srci)
        pltpu.sync_copy(dst_hbm.at[pl.ds(base, _CH)], dsti.at[0])
        return pltpu.async_copy(h_hbm.at[srci], rows, semg)

    def weights(buf, ngroups=_CH // _L):
        srci, dsti, s_v, _, _, _ = buf
        for g in range(ngroups):
            sv = srci[pl.ds(g * _L, _L)]
            dv = dsti[0, pl.ds(g * _L, _L)]
            u = plsc.load_gather(asrc_v, [sv]) + plsc.load_gather(adst_v, [dv])
            e = jnp.maximum(u, 0.2 * u)
            s = jnp.exp(e - creg)
            s_v[pl.ds(g * _L, _L)] = s
            plsc.addupdate_scatter(den_v, [dv >> 7, dv & 127], s)

    def scale(buf):
        _, _, s_v, rows, _, _ = buf

        @pl.loop(0, _CH)
        def _(r):
            sp = plsc.load_gather(s_v, [jnp.full((_L,), r, jnp.int32)])
            for j in range(_D // _L):
                sl = (r, pl.ds(j * _L, _L))
                rows[sl] = rows[sl] * sp

    def wait_gather(buf):
        srci, _, _, rows, semg, _ = buf
        pltpu.make_async_copy(h_hbm.at[srci], rows, semg).wait()

    def start_scatter(buf):
        _, dsti, _, rows, _, sems = buf
        pltpu.async_copy(rows, acc_sh.at[dsti.at[0]], sems, add=True)

    def wait_scatter(buf):
        _, dsti, _, rows, _, sems = buf
        pltpu.make_async_copy(rows, acc_sh.at[dsti.at[0]], sems).wait()

    fetch(basew, bufs[0])
    weights(bufs[0])

    @pl.loop(0, _NCHM // 3 - 1)
    def _(i):
        for b in range(3):
            k = i * 3 + b
            cur = bufs[b]
            nxt = bufs[(b + 1) % 3]

            @pl.when(k >= 2)
            def _():
                wait_scatter(nxt)
            fetch(basew + (k + 1) * _CH, nxt)
            wait_gather(cur)
            scale(cur)
            start_scatter(cur)
            weights(nxt)

    for k in range(_NCHM - 3, _NCHM):
        b = k % 3
        cur = bufs[b]
        nxt = bufs[(b + 1) % 3]
        if k + 1 < _NCHM:
            wait_scatter(nxt)
            fetch(basew + (k + 1) * _CH, nxt)
        wait_gather(cur)
        scale(cur)
        start_scatter(cur)
        if k + 1 < _NCHM:
            weights(nxt)
    for b in range(3):
        wait_scatter(bufs[b])

    @pl.when(wid < 16)
    def _():
        cp = fetch(_REM_BASE + wid * _CH, bufs[0])
        weights(bufs[0])
        cp.wait()
        scale(bufs[0])
        _, dsti, _, rows, _, _ = bufs[0]
        pltpu.sync_copy(rows, acc_sh.at[dsti.at[0]], add=True)

    pltpu.sync_copy(den_v, den_sh.at[idn_v.at[0]], add=True)
    plsc.subcore_barrier()

    pltpu.sync_copy(acc_sh.at[pl.ds(sid * _RPS, _RPS)],
                    parts_hbm.at[cid].at[pl.ds(sid * _RPS, _RPS)])

    @pl.when(sid == 0)
    def _():
        pltpu.sync_copy(den_sh, denp_hbm.at[cid])


_sc_params = pltpu.CompilerParams()
if "needs_layout_passes" in pltpu.CompilerParams.__dataclass_fields__:
    _sc_params = dataclasses.replace(_sc_params, needs_layout_passes=False)

_sc1 = pl.kernel(
    _sc1_body,
    out_type=[
        jax.ShapeDtypeStruct((_NC, _NP, _D), jnp.float32),
        jax.ShapeDtypeStruct((_NC, _DRW, _D), jnp.float32),
    ],
    mesh=plsc.VectorSubcoreMesh(core_axis_name="c", subcore_axis_name="s"),
    compiler_params=_sc_params,
    scratch_types=[
        pltpu.VMEM((_AT,), jnp.float32),
        pltpu.VMEM((_AT,), jnp.float32),
        pltpu.VMEM((_L,), jnp.float32),
        pltpu.VMEM((_CH,), jnp.int32),
        pltpu.VMEM((_CH,), jnp.int32),
        pltpu.VMEM((_CH,), jnp.int32),
        pltpu.VMEM((1, _CH), jnp.int32),
        pltpu.VMEM((1, _CH), jnp.int32),
        pltpu.VMEM((1, _CH), jnp.int32),
        pltpu.VMEM((_CH,), jnp.float32),
        pltpu.VMEM((_CH,), jnp.float32),
        pltpu.VMEM((_CH,), jnp.float32),
        pltpu.VMEM((_CH, _D), jnp.float32),
        pltpu.VMEM((_CH, _D), jnp.float32),
        pltpu.VMEM((_CH, _D), jnp.float32),
        pltpu.VMEM((_DRW, _D), jnp.float32),
        pltpu.VMEM((1, _DRW), jnp.int32),
        pltpu.VMEM_SHARED((_NP, _D), jnp.float32),
        pltpu.VMEM_SHARED((_DRW, _D), jnp.float32),
        pltpu.SemaphoreType.DMA,
        pltpu.SemaphoreType.DMA,
        pltpu.SemaphoreType.DMA,
        pltpu.SemaphoreType.DMA,
        pltpu.SemaphoreType.DMA,
        pltpu.SemaphoreType.DMA,
    ],
)


def _sc2_body(parts_hbm, denp_hbm, h_hbm, ss_hbm, outd_hbm,
              p0_v, p1_v, h_v, ss_v, d0_v, d1_v, sem):
    cid = lax.axis_index("c")
    sid = lax.axis_index("s")
    wid = cid * _NS + sid

    pltpu.sync_copy(denp_hbm.at[0], d0_v)
    pltpu.sync_copy(denp_hbm.at[1], d1_v)

    nblk = jnp.where(wid < 15, 3, 2)
    blk0 = jnp.where(wid < 15, 3 * wid, 2 * wid + 15)

    for q in range(3):
        @pl.when(q < nblk)
        def _():
            j = blk0 + q
            base = j * _D
            pltpu.sync_copy(parts_hbm.at[0].at[pl.ds(base, _D)], p0_v)
            pltpu.sync_copy(parts_hbm.at[1].at[pl.ds(base, _D)], p1_v)
            pltpu.sync_copy(h_hbm.at[pl.ds(base, _D)], h_v)
            pltpu.sync_copy(ss_hbm.at[pl.ds(base, _D)], ss_v)

            @pl.loop(0, _D)
            def _(r):
                jrow = jnp.full((_L,), j, jnp.int32)
                rcol = jnp.full((_L,), r, jnp.int32)
                den = (plsc.load_gather(d0_v, [jrow, rcol]) +
                       plsc.load_gather(d1_v, [jrow, rcol]))
                ssr = plsc.load_gather(ss_v, [rcol])
                rec = 1.0 / (den + ssr + 1e-16)
                for jj in range(_D // _L):
                    sl = (r, pl.ds(jj * _L, _L))
                    p0_v[sl] = (p0_v[sl] + p1_v[sl] + ssr * h_v[sl]) * rec

            pltpu.sync_copy(p0_v, outd_hbm.at[pl.ds(base, _D)])


_sc2 = pl.kernel(
    _sc2_body,
    out_type=jax.ShapeDtypeStruct((_NP, _D), jnp.float32),
    mesh=plsc.VectorSubcoreMesh(core_axis_name="c", subcore_axis_name="s"),
    compiler_params=_sc_params,
    scratch_types=[
        pltpu.VMEM((_D, _D), jnp.float32),
        pltpu.VMEM((_D, _D), jnp.float32),
        pltpu.VMEM((_D, _D), jnp.float32),
        pltpu.VMEM((_D,), jnp.float32),
        pltpu.VMEM((_DRW, _D), jnp.float32),
        pltpu.VMEM((_DRW, _D), jnp.float32),
        pltpu.SemaphoreType.DMA,
    ],
)


def _tc2_body(outd_ref, b_ref, o_ref):
    o_ref[...] = jnp.tanh(outd_ref[0:_N, :] + b_ref[...])


_tc2 = pl.pallas_call(
    _tc2_body,
    out_shape=jax.ShapeDtypeStruct((_N, _D), jnp.float32),
)


@jax.jit
def kernel(x, edge_index, W, att_src, att_dst, bias):
    src = edge_index[0]
    dst = edge_index[1]
    hp, asrc, adst, ssp, c = _tc1(x, W, att_src.reshape(_D, 1),
                                  att_dst.reshape(_D, 1))
    cvec = jnp.broadcast_to(c.reshape(1), (_L,))
    zrows = jnp.zeros((_RPS, _D), jnp.float32)
    parts, denp = _sc1(hp, src, dst, asrc.reshape(_NP), adst.reshape(_NP),
                       cvec, zrows)
    outd = _sc2(parts, denp, hp, ssp.reshape(_NP))
    return _tc2(outd, bias.reshape(1, _D))


# speedup vs baseline: 37.2816x; 1.5648x over previous
import dataclasses

import jax
import jax.numpy as jnp
from jax import lax
from jax.experimental import pallas as pl
from jax.experimental.pallas import tpu as pltpu
from jax.experimental.pallas import tpu_sc as plsc

_N = 10000
_E = 320000
_D = 128
_NC = 2
_NS = 16
_NW = _NC * _NS
_CH = 32
_NCHM = 312
_REM_BASE = _NCHM * _CH * _NW
_EPW = _NCHM * _CH
_NP = 10112
_RPS = _NP // _NS
_AT = 10000
_DRW = 80
_NB = _NP // _D
_L = 16


def _tc1_body(x_ref, w_ref, as_ref, ad_ref, h_ref, asrc_ref, adst_ref,
              ss_ref, c_ref):
    h = lax.dot_general(x_ref[...], w_ref[...], (((1,), (0,)), ((), ())),
                        preferred_element_type=jnp.float32)
    h_ref[0:_N, :] = h
    h_ref[_N:_NP, :] = jnp.zeros((_NP - _N, _D), jnp.float32)
    a_s = lax.dot_general(h, as_ref[...], (((1,), (0,)), ((), ())),
                          preferred_element_type=jnp.float32)
    a_d = lax.dot_general(h, ad_ref[...], (((1,), (0,)), ((), ())),
                          preferred_element_type=jnp.float32)
    asrc_ref[0:_N, :] = a_s
    asrc_ref[_N:_NP, :] = jnp.zeros((_NP - _N, 1), jnp.float32)
    adst_ref[0:_N, :] = a_d
    adst_ref[_N:_NP, :] = jnp.zeros((_NP - _N, 1), jnp.float32)
    t = jnp.max(a_s) + jnp.max(a_d)
    c = jnp.maximum(t, 0.2 * t)
    u = a_s + a_d
    ss_ref[0:_N, :] = jnp.exp(jnp.maximum(u, 0.2 * u) - c)
    ss_ref[_N:_NP, :] = jnp.ones((_NP - _N, 1), jnp.float32)
    c_ref[...] = jnp.broadcast_to(c, (1, 1))


_tc1 = pl.pallas_call(
    _tc1_body,
    out_shape=[
        jax.ShapeDtypeStruct((_NP, _D), jnp.float32),
        jax.ShapeDtypeStruct((_NP, 1), jnp.float32),
        jax.ShapeDtypeStruct((_NP, 1), jnp.float32),
        jax.ShapeDtypeStruct((_NP, 1), jnp.float32),
        jax.ShapeDtypeStruct((1, 1), jnp.float32),
    ],
)


def _sc1_body(h_hbm, src_hbm, dst_hbm, asrc_hbm, adst_hbm, cvec_hbm, zrows_hbm,
              parts_hbm, denp_hbm,
              asrc_v, adst_v, c_v,
              srci0, srci1, srci2, srci3, dsti0, dsti1, dsti2, dsti3,
              sv0, sv1, sv2, sv3, rows0, rows1, rows2, den_v, idn_v,
              acc_sh, den_sh,
              semi0, semi1, semi2, semi3, semg0, semg1, semg2,
              sems0, sems1, sems2):
    cid = lax.axis_index("c")
    sid = lax.axis_index("s")
    wid = cid * _NS + sid

    pltpu.sync_copy(zrows_hbm, acc_sh.at[pl.ds(sid * _RPS, _RPS)])
    pltpu.sync_copy(zrows_hbm.at[pl.ds(0, _DRW // _NS)],
                    den_sh.at[pl.ds(sid * (_DRW // _NS), _DRW // _NS)])
    pltpu.sync_copy(zrows_hbm.at[pl.ds(0, _DRW)], den_v)
    pltpu.sync_copy(asrc_hbm.at[pl.ds(0, _AT)], asrc_v)
    pltpu.sync_copy(adst_hbm.at[pl.ds(0, _AT)], adst_v)
    pltpu.sync_copy(cvec_hbm, c_v)
    for g in range(_DRW // _L):
        idn_v[0, pl.ds(g * _L, _L)] = g * _L + lax.iota(jnp.int32, _L)
    plsc.subcore_barrier()

    creg = c_v[...]
    basew = wid * _EPW
    ibufs = ((srci0, dsti0, sv0, semi0), (srci1, dsti1, sv1, semi1),
             (srci2, dsti2, sv2, semi2), (srci3, dsti3, sv3, semi3))
    rbufs = ((rows0, semg0, sems0), (rows1, semg1, sems1),
             (rows2, semg2, sems2))

    def idx_start(base, ib):
        srci, dsti, _, semi = ib
        pltpu.async_copy(src_hbm.at[pl.ds(base, _CH)], srci, semi)
        pltpu.async_copy(dst_hbm.at[pl.ds(base, _CH)], dsti.at[0], semi)

    def idx_wait(base, ib):
        srci, dsti, _, semi = ib
        pltpu.make_async_copy(src_hbm.at[pl.ds(base, _CH)], srci, semi).wait()
        pltpu.make_async_copy(dst_hbm.at[pl.ds(base, _CH)], dsti.at[0],
                              semi).wait()

    def gather_start(ib, rb):
        pltpu.async_copy(h_hbm.at[ib[0]], rb[0], rb[1])

    def gather_wait(ib, rb):
        pltpu.make_async_copy(h_hbm.at[ib[0]], rb[0], rb[1]).wait()

    def scatter_start(ib, rb):
        pltpu.async_copy(rb[0], acc_sh.at[ib[1].at[0]], rb[2], add=True)

    def scatter_wait(ib, rb):
        pltpu.make_async_copy(rb[0], acc_sh.at[ib[1].at[0]], rb[2]).wait()

    def weights(ib):
        srci, dsti, s_v, _ = ib
        for g in range(_CH // _L):
            sv = srci[pl.ds(g * _L, _L)]
            dv = dsti[0, pl.ds(g * _L, _L)]
            u = plsc.load_gather(asrc_v, [sv]) + plsc.load_gather(adst_v, [dv])
            e = jnp.maximum(u, 0.2 * u)
            s = jnp.exp(e - creg)
            s_v[pl.ds(g * _L, _L)] = s
            plsc.addupdate_scatter(den_v, [dv >> 7, dv & 127], s)

    def scale(ib, rb):
        s_v = ib[2]
        rows = rb[0]

        @pl.loop(0, _CH)
        def _(r):
            sp = plsc.load_gather(s_v, [jnp.full((_L,), r, jnp.int32)])
            for j in range(_D // _L):
                sl = (r, pl.ds(j * _L, _L))
                rows[sl] = rows[sl] * sp

    idx_start(basew, ibufs[0])
    idx_wait(basew, ibufs[0])
    gather_start(ibufs[0], rbufs[0])
    idx_start(basew + _CH, ibufs[1])
    weights(ibufs[0])

    @pl.loop(0, _NCHM // 12)
    def _(i):
        for b in range(12):
            k = i * 12 + b
            ib_cur, rb_cur = ibufs[b % 4], rbufs[b % 3]
            ib_nxt, rb_nxt = ibufs[(b + 1) % 4], rbufs[(b + 1) % 3]
            ib_pre = ibufs[(b + 2) % 4]

            if b >= 2:
                scatter_wait(ib_pre, rb_nxt)
            else:
                @pl.when(k >= 2)
                def _():
                    scatter_wait(ib_pre, rb_nxt)

            def prep_next():
                idx_wait(basew + (k + 1) * _CH, ib_nxt)
                gather_start(ib_nxt, rb_nxt)

            def prefetch_idx():
                idx_start(basew + (k + 2) * _CH, ib_pre)

            if b == 11:
                @pl.when(k + 1 < _NCHM)
                def _():
                    prep_next()
            else:
                prep_next()
            if b >= 10:
                @pl.when(k + 2 < _NCHM)
                def _():
                    prefetch_idx()
            else:
                prefetch_idx()

            gather_wait(ib_cur, rb_cur)
            scale(ib_cur, rb_cur)
            scatter_start(ib_cur, rb_cur)

            if b == 11:
                @pl.when(k + 1 < _NCHM)
                def _():
                    weights(ib_nxt)
            else:
                weights(ib_nxt)

    scatter_wait(ibufs[310 % 4], rbufs[310 % 3])
    scatter_wait(ibufs[311 % 4], rbufs[311 % 3])

    @pl.when(wid < 16)
    def _():
        base = _REM_BASE + wid * _CH
        idx_start(base, ibufs[0])
        idx_wait(base, ibufs[0])
        gather_start(ibufs[0], rbufs[0])
        weights(ibufs[0])
        gather_wait(ibufs[0], rbufs[0])
        scale(ibufs[0], rbufs[0])
        pltpu.sync_copy(rbufs[0][0], acc_sh.at[ibufs[0][1].at[0]], add=True)

    pltpu.sync_copy(den_v, den_sh.at[idn_v.at[0]], add=True)
    plsc.subcore_barrier()

    pltpu.sync_copy(acc_sh.at[pl.ds(sid * _RPS, _RPS)],
                    parts_hbm.at[cid].at[pl.ds(sid * _RPS, _RPS)])

    @pl.when(sid == 0)
    def _():
        pltpu.sync_copy(den_sh, denp_hbm.at[cid])


_sc_params = pltpu.CompilerParams()
if "needs_layout_passes" in pltpu.CompilerParams.__dataclass_fields__:
    _sc_params = dataclasses.replace(_sc_params, needs_layout_passes=False)

_sc1 = pl.kernel(
    _sc1_body,
    out_type=[
        jax.ShapeDtypeStruct((_NC, _NP, _D), jnp.float32),
        jax.ShapeDtypeStruct((_NC, _DRW, _D), jnp.float32),
    ],
    mesh=plsc.VectorSubcoreMesh(core_axis_name="c", subcore_axis_name="s"),
    compiler_params=_sc_params,
    scratch_types=[
        pltpu.VMEM((_AT,), jnp.float32),
        pltpu.VMEM((_AT,), jnp.float32),
        pltpu.VMEM((_L,), jnp.float32),
        pltpu.VMEM((_CH,), jnp.int32),
        pltpu.VMEM((_CH,), jnp.int32),
        pltpu.VMEM((_CH,), jnp.int32),
        pltpu.VMEM((_CH,), jnp.int32),
        pltpu.VMEM((1, _CH), jnp.int32),
        pltpu.VMEM((1, _CH), jnp.int32),
        pltpu.VMEM((1, _CH), jnp.int32),
        pltpu.VMEM((1, _CH), jnp.int32),
        pltpu.VMEM((_CH,), jnp.float32),
        pltpu.VMEM((_CH,), jnp.float32),
        pltpu.VMEM((_CH,), jnp.float32),
        pltpu.VMEM((_CH,), jnp.float32),
        pltpu.VMEM((_CH, _D), jnp.float32),
        pltpu.VMEM((_CH, _D), jnp.float32),
        pltpu.VMEM((_CH, _D), jnp.float32),
        pltpu.VMEM((_DRW, _D), jnp.float32),
        pltpu.VMEM((1, _DRW), jnp.int32),
        pltpu.VMEM_SHARED((_NP, _D), jnp.float32),
        pltpu.VMEM_SHARED((_DRW, _D), jnp.float32),
        pltpu.SemaphoreType.DMA,
        pltpu.SemaphoreType.DMA,
        pltpu.SemaphoreType.DMA,
        pltpu.SemaphoreType.DMA,
        pltpu.SemaphoreType.DMA,
        pltpu.SemaphoreType.DMA,
        pltpu.SemaphoreType.DMA,
        pltpu.SemaphoreType.DMA,
        pltpu.SemaphoreType.DMA,
        pltpu.SemaphoreType.DMA,
    ],
)


def _sc2_body(parts_hbm, denp_hbm, h_hbm, ss_hbm, outd_hbm,
              p0_v, p1_v, h_v, ss_v, d0_v, d1_v, sem):
    cid = lax.axis_index("c")
    sid = lax.axis_index("s")
    wid = cid * _NS + sid

    pltpu.sync_copy(denp_hbm.at[0], d0_v)
    pltpu.sync_copy(denp_hbm.at[1], d1_v)

    nblk = jnp.where(wid < 15, 3, 2)
    blk0 = jnp.where(wid < 15, 3 * wid, 2 * wid + 15)

    for q in range(3):
        @pl.when(q < nblk)
        def _():
            j = blk0 + q
            base = j * _D
            pltpu.sync_copy(parts_hbm.at[0].at[pl.ds(base, _D)], p0_v)
            pltpu.sync_copy(parts_hbm.at[1].at[pl.ds(base, _D)], p1_v)
            pltpu.sync_copy(h_hbm.at[pl.ds(base, _D)], h_v)
            pltpu.sync_copy(ss_hbm.at[pl.ds(base, _D)], ss_v)

            @pl.loop(0, _D)
            def _(r):
                jrow = jnp.full((_L,), j, jnp.int32)
                rcol = jnp.full((_L,), r, jnp.int32)
                den = (plsc.load_gather(d0_v, [jrow, rcol]) +
                       plsc.load_gather(d1_v, [jrow, rcol]))
                ssr = plsc.load_gather(ss_v, [rcol])
                rec = 1.0 / (den + ssr + 1e-16)
                for jj in range(_D // _L):
                    sl = (r, pl.ds(jj * _L, _L))
                    p0_v[sl] = (p0_v[sl] + p1_v[sl] + ssr * h_v[sl]) * rec

            pltpu.sync_copy(p0_v, outd_hbm.at[pl.ds(base, _D)])


_sc2 = pl.kernel(
    _sc2_body,
    out_type=jax.ShapeDtypeStruct((_NP, _D), jnp.float32),
    mesh=plsc.VectorSubcoreMesh(core_axis_name="c", subcore_axis_name="s"),
    compiler_params=_sc_params,
    scratch_types=[
        pltpu.VMEM((_D, _D), jnp.float32),
        pltpu.VMEM((_D, _D), jnp.float32),
        pltpu.VMEM((_D, _D), jnp.float32),
        pltpu.VMEM((_D,), jnp.float32),
        pltpu.VMEM((_DRW, _D), jnp.float32),
        pltpu.VMEM((_DRW, _D), jnp.float32),
        pltpu.SemaphoreType.DMA,
    ],
)


def _tc2_body(outd_ref, b_ref, o_ref):
    o_ref[...] = jnp.tanh(outd_ref[0:_N, :] + b_ref[...])


_tc2 = pl.pallas_call(
    _tc2_body,
    out_shape=jax.ShapeDtypeStruct((_N, _D), jnp.float32),
)


@jax.jit
def kernel(x, edge_index, W, att_src, att_dst, bias):
    src = edge_index[0]
    dst = edge_index[1]
    hp, asrc, adst, ssp, c = _tc1(x, W, att_src.reshape(_D, 1),
                                  att_dst.reshape(_D, 1))
    cvec = jnp.broadcast_to(c.reshape(1), (_L,))
    zrows = jnp.zeros((_RPS, _D), jnp.float32)
    parts, denp = _sc1(hp, src, dst, asrc.reshape(_NP), adst.reshape(_NP),
                       cvec, zrows)
    outd = _sc2(parts, denp, hp, ssp.reshape(_NP))
    return _tc2(outd, bias.reshape(1, _D))


# speedup vs baseline: 39.2082x; 1.0517x over previous
import dataclasses

import jax
import jax.numpy as jnp
from jax import lax
from jax.experimental import pallas as pl
from jax.experimental.pallas import tpu as pltpu
from jax.experimental.pallas import tpu_sc as plsc

_N = 10000
_E = 320000
_D = 128
_NC = 2
_NS = 16
_NW = _NC * _NS
_CH = 32
_NCHM = 312
_REM_BASE = _NCHM * _CH * _NW
_EPW = _NCHM * _CH
_NP = 10112
_RPS = _NP // _NS
_AT = 10000
_DRW = 80
_NB = _NP // _D
_L = 16


def _tc1_body(x_ref, w_ref, as_ref, ad_ref, h_ref, asrc_ref, adst_ref,
              ss_ref, c_ref):
    h = lax.dot_general(x_ref[...], w_ref[...], (((1,), (0,)), ((), ())),
                        preferred_element_type=jnp.float32)
    h_ref[0:_N, :] = h
    h_ref[_N:_NP, :] = jnp.zeros((_NP - _N, _D), jnp.float32)
    a_s = lax.dot_general(h, as_ref[...], (((1,), (0,)), ((), ())),
                          preferred_element_type=jnp.float32)
    a_d = lax.dot_general(h, ad_ref[...], (((1,), (0,)), ((), ())),
                          preferred_element_type=jnp.float32)
    asrc_ref[0:_N, :] = a_s
    asrc_ref[_N:_NP, :] = jnp.zeros((_NP - _N, 1), jnp.float32)
    adst_ref[0:_N, :] = a_d
    adst_ref[_N:_NP, :] = jnp.zeros((_NP - _N, 1), jnp.float32)
    t = jnp.max(a_s) + jnp.max(a_d)
    c = jnp.maximum(t, 0.2 * t)
    u = a_s + a_d
    ss_ref[0:_N, :] = jnp.exp(jnp.maximum(u, 0.2 * u) - c)
    ss_ref[_N:_NP, :] = jnp.ones((_NP - _N, 1), jnp.float32)
    c_ref[...] = jnp.broadcast_to(c, (1, 1))


_tc1 = pl.pallas_call(
    _tc1_body,
    out_shape=[
        jax.ShapeDtypeStruct((_NP, _D), jnp.float32),
        jax.ShapeDtypeStruct((_NP, 1), jnp.float32),
        jax.ShapeDtypeStruct((_NP, 1), jnp.float32),
        jax.ShapeDtypeStruct((_NP, 1), jnp.float32),
        jax.ShapeDtypeStruct((1, 1), jnp.float32),
    ],
)


def _sc1_body(h_hbm, src_hbm, dst_hbm, asrc_hbm, adst_hbm, cvec_hbm, zrows_hbm,
              parts_hbm, denp_hbm,
              asrc_v, adst_v, c_v,
              srci0, srci1, srci2, srci3, dsti0, dsti1, dsti2, dsti3,
              sv0, sv1, sv2, sv3, rows0, rows1, rows2, den_v, idn_v,
              acc_sh, den_sh,
              semi0, semi1, semi2, semi3, semg0, semg1, semg2,
              sems0, sems1, sems2):
    cid = lax.axis_index("c")
    sid = lax.axis_index("s")
    wid = cid * _NS + sid

    pltpu.sync_copy(zrows_hbm, acc_sh.at[pl.ds(sid * _RPS, _RPS)])
    pltpu.sync_copy(zrows_hbm.at[pl.ds(0, _DRW // _NS)],
                    den_sh.at[pl.ds(sid * (_DRW // _NS), _DRW // _NS)])
    pltpu.sync_copy(zrows_hbm.at[pl.ds(0, _DRW)], den_v)
    pltpu.sync_copy(asrc_hbm.at[pl.ds(0, _AT)], asrc_v)
    pltpu.sync_copy(adst_hbm.at[pl.ds(0, _AT)], adst_v)
    pltpu.sync_copy(cvec_hbm, c_v)
    for g in range(_DRW // _L):
        idn_v[0, pl.ds(g * _L, _L)] = g * _L + lax.iota(jnp.int32, _L)
    plsc.subcore_barrier()

    creg = c_v[...]
    basew = wid * _EPW
    ibufs = ((srci0, dsti0, sv0, semi0), (srci1, dsti1, sv1, semi1),
             (srci2, dsti2, sv2, semi2), (srci3, dsti3, sv3, semi3))
    rbufs = ((rows0, semg0, sems0), (rows1, semg1, sems1),
             (rows2, semg2, sems2))

    def idx_start(base, ib):
        srci, dsti, _, semi = ib
        pltpu.async_copy(src_hbm.at[pl.ds(base, _CH)], srci, semi)
        pltpu.async_copy(dst_hbm.at[pl.ds(base, _CH)], dsti.at[0], semi)

    def idx_wait(base, ib):
        srci, dsti, _, semi = ib
        pltpu.make_async_copy(src_hbm.at[pl.ds(base, _CH)], srci, semi).wait()
        pltpu.make_async_copy(dst_hbm.at[pl.ds(base, _CH)], dsti.at[0],
                              semi).wait()

    def gather_start(ib, rb):
        pltpu.async_copy(h_hbm.at[ib[0]], rb[0], rb[1])

    def gather_wait(ib, rb):
        pltpu.make_async_copy(h_hbm.at[ib[0]], rb[0], rb[1]).wait()

    def scatter_start(ib, rb):
        pltpu.async_copy(rb[0], acc_sh.at[ib[1].at[0]], rb[2], add=True)

    def scatter_wait(ib, rb):
        pltpu.make_async_copy(rb[0], acc_sh.at[ib[1].at[0]], rb[2]).wait()

    def weights(ib):
        srci, dsti, s_v, _ = ib
        for g in range(_CH // _L):
            sv = srci[pl.ds(g * _L, _L)]
            dv = dsti[0, pl.ds(g * _L, _L)]
            u = plsc.load_gather(asrc_v, [sv]) + plsc.load_gather(adst_v, [dv])
            e = jnp.maximum(u, 0.2 * u)
            s = jnp.exp(e - creg)
            s_v[pl.ds(g * _L, _L)] = s
            plsc.addupdate_scatter(den_v, [dv >> 7, dv & 127], s)

    def scale(ib, rb):
        s_v = ib[2]
        rows = rb[0]

        @plsc.parallel_loop(0, _CH, unroll=4)
        def _(r):
            sp = plsc.load_gather(s_v, [jnp.full((_L,), r, jnp.int32)])
            for j in range(_D // _L):
                sl = (r, pl.ds(j * _L, _L))
                rows[sl] = rows[sl] * sp

    idx_start(basew, ibufs[0])
    idx_wait(basew, ibufs[0])
    gather_start(ibufs[0], rbufs[0])
    idx_start(basew + _CH, ibufs[1])
    weights(ibufs[0])

    @pl.loop(0, _NCHM // 12)
    def _(i):
        for b in range(12):
            k = i * 12 + b
            ib_cur, rb_cur = ibufs[b % 4], rbufs[b % 3]
            ib_nxt, rb_nxt = ibufs[(b + 1) % 4], rbufs[(b + 1) % 3]
            ib_pre = ibufs[(b + 2) % 4]

            if b >= 2:
                scatter_wait(ib_pre, rb_nxt)
            else:
                @pl.when(k >= 2)
                def _():
                    scatter_wait(ib_pre, rb_nxt)

            def prep_next():
                idx_wait(basew + (k + 1) * _CH, ib_nxt)
                gather_start(ib_nxt, rb_nxt)

            def prefetch_idx():
                idx_start(basew + (k + 2) * _CH, ib_pre)

            if b == 11:
                @pl.when(k + 1 < _NCHM)
                def _():
                    prep_next()
            else:
                prep_next()
            if b >= 10:
                @pl.when(k + 2 < _NCHM)
                def _():
                    prefetch_idx()
            else:
                prefetch_idx()

            gather_wait(ib_cur, rb_cur)
            scale(ib_cur, rb_cur)
            scatter_start(ib_cur, rb_cur)

            if b == 11:
                @pl.when(k + 1 < _NCHM)
                def _():
                    weights(ib_nxt)
            else:
                weights(ib_nxt)

    scatter_wait(ibufs[310 % 4], rbufs[310 % 3])
    scatter_wait(ibufs[311 % 4], rbufs[311 % 3])

    @pl.when(wid < 16)
    def _():
        base = _REM_BASE + wid * _CH
        idx_start(base, ibufs[0])
        idx_wait(base, ibufs[0])
        gather_start(ibufs[0], rbufs[0])
        weights(ibufs[0])
        gather_wait(ibufs[0], rbufs[0])
        scale(ibufs[0], rbufs[0])
        pltpu.sync_copy(rbufs[0][0], acc_sh.at[ibufs[0][1].at[0]], add=True)

    pltpu.sync_copy(den_v, den_sh.at[idn_v.at[0]], add=True)
    plsc.subcore_barrier()

    pltpu.sync_copy(acc_sh.at[pl.ds(sid * _RPS, _RPS)],
                    parts_hbm.at[cid].at[pl.ds(sid * _RPS, _RPS)])

    @pl.when(sid == 0)
    def _():
        pltpu.sync_copy(den_sh, denp_hbm.at[cid])


_sc_params = pltpu.CompilerParams()
if "needs_layout_passes" in pltpu.CompilerParams.__dataclass_fields__:
    _sc_params = dataclasses.replace(_sc_params, needs_layout_passes=False)

_sc1 = pl.kernel(
    _sc1_body,
    out_type=[
        jax.ShapeDtypeStruct((_NC, _NP, _D), jnp.float32),
        jax.ShapeDtypeStruct((_NC, _DRW, _D), jnp.float32),
    ],
    mesh=plsc.VectorSubcoreMesh(core_axis_name="c", subcore_axis_name="s"),
    compiler_params=_sc_params,
    scratch_types=[
        pltpu.VMEM((_AT,), jnp.float32),
        pltpu.VMEM((_AT,), jnp.float32),
        pltpu.VMEM((_L,), jnp.float32),
        pltpu.VMEM((_CH,), jnp.int32),
        pltpu.VMEM((_CH,), jnp.int32),
        pltpu.VMEM((_CH,), jnp.int32),
        pltpu.VMEM((_CH,), jnp.int32),
        pltpu.VMEM((1, _CH), jnp.int32),
        pltpu.VMEM((1, _CH), jnp.int32),
        pltpu.VMEM((1, _CH), jnp.int32),
        pltpu.VMEM((1, _CH), jnp.int32),
        pltpu.VMEM((_CH,), jnp.float32),
        pltpu.VMEM((_CH,), jnp.float32),
        pltpu.VMEM((_CH,), jnp.float32),
        pltpu.VMEM((_CH,), jnp.float32),
        pltpu.VMEM((_CH, _D), jnp.float32),
        pltpu.VMEM((_CH, _D), jnp.float32),
        pltpu.VMEM((_CH, _D), jnp.float32),
        pltpu.VMEM((_DRW, _D), jnp.float32),
        pltpu.VMEM((1, _DRW), jnp.int32),
        pltpu.VMEM_SHARED((_NP, _D), jnp.float32),
        pltpu.VMEM_SHARED((_DRW, _D), jnp.float32),
        pltpu.SemaphoreType.DMA,
        pltpu.SemaphoreType.DMA,
        pltpu.SemaphoreType.DMA,
        pltpu.SemaphoreType.DMA,
        pltpu.SemaphoreType.DMA,
        pltpu.SemaphoreType.DMA,
        pltpu.SemaphoreType.DMA,
        pltpu.SemaphoreType.DMA,
        pltpu.SemaphoreType.DMA,
        pltpu.SemaphoreType.DMA,
    ],
)


def _sc2_body(parts_hbm, denp_hbm, h_hbm, ss_hbm, outd_hbm,
              p0_v, p1_v, h_v, ss_v, d0_v, d1_v, sem):
    cid = lax.axis_index("c")
    sid = lax.axis_index("s")
    wid = cid * _NS + sid

    pltpu.sync_copy(denp_hbm.at[0], d0_v)
    pltpu.sync_copy(denp_hbm.at[1], d1_v)

    nblk = jnp.where(wid < 15, 3, 2)
    blk0 = jnp.where(wid < 15, 3 * wid, 2 * wid + 15)

    for q in range(3):
        @pl.when(q < nblk)
        def _():
            j = blk0 + q
            base = j * _D
            pltpu.sync_copy(parts_hbm.at[0].at[pl.ds(base, _D)], p0_v)
            pltpu.sync_copy(parts_hbm.at[1].at[pl.ds(base, _D)], p1_v)
            pltpu.sync_copy(h_hbm.at[pl.ds(base, _D)], h_v)
            pltpu.sync_copy(ss_hbm.at[pl.ds(base, _D)], ss_v)

            @pl.loop(0, _D)
            def _(r):
                jrow = jnp.full((_L,), j, jnp.int32)
                rcol = jnp.full((_L,), r, jnp.int32)
                den = (plsc.load_gather(d0_v, [jrow, rcol]) +
                       plsc.load_gather(d1_v, [jrow, rcol]))
                ssr = plsc.load_gather(ss_v, [rcol])
                rec = 1.0 / (den + ssr + 1e-16)
                for jj in range(_D // _L):
                    sl = (r, pl.ds(jj * _L, _L))
                    p0_v[sl] = (p0_v[sl] + p1_v[sl] + ssr * h_v[sl]) * rec

            pltpu.sync_copy(p0_v, outd_hbm.at[pl.ds(base, _D)])


_sc2 = pl.kernel(
    _sc2_body,
    out_type=jax.ShapeDtypeStruct((_NP, _D), jnp.float32),
    mesh=plsc.VectorSubcoreMesh(core_axis_name="c", subcore_axis_name="s"),
    compiler_params=_sc_params,
    scratch_types=[
        pltpu.VMEM((_D, _D), jnp.float32),
        pltpu.VMEM((_D, _D), jnp.float32),
        pltpu.VMEM((_D, _D), jnp.float32),
        pltpu.VMEM((_D,), jnp.float32),
        pltpu.VMEM((_DRW, _D), jnp.float32),
        pltpu.VMEM((_DRW, _D), jnp.float32),
        pltpu.SemaphoreType.DMA,
    ],
)


def _tc2_body(outd_ref, b_ref, o_ref):
    o_ref[...] = jnp.tanh(outd_ref[0:_N, :] + b_ref[...])


_tc2 = pl.pallas_call(
    _tc2_body,
    out_shape=jax.ShapeDtypeStruct((_N, _D), jnp.float32),
)


@jax.jit
def kernel(x, edge_index, W, att_src, att_dst, bias):
    src = edge_index[0]
    dst = edge_index[1]
    hp, asrc, adst, ssp, c = _tc1(x, W, att_src.reshape(_D, 1),
                                  att_dst.reshape(_D, 1))
    cvec = jnp.broadcast_to(c.reshape(1), (_L,))
    zrows = jnp.zeros((_RPS, _D), jnp.float32)
    parts, denp = _sc1(hp, src, dst, asrc.reshape(_NP), adst.reshape(_NP),
                       cvec, zrows)
    outd = _sc2(parts, denp, hp, ssp.reshape(_NP))
    return _tc2(outd, bias.reshape(1, _D))


# speedup vs baseline: 41.5540x; 1.0598x over previous
import dataclasses

import jax
import jax.numpy as jnp
from jax import lax
from jax.experimental import pallas as pl
from jax.experimental.pallas import tpu as pltpu
from jax.experimental.pallas import tpu_sc as plsc

_N = 10000
_E = 320000
_D = 128
_NC = 2
_NS = 16
_NW = _NC * _NS
_CH = 32
_NCHM = 312
_REM_BASE = _NCHM * _CH * _NW
_EPW = _NCHM * _CH
_NP = 10112
_RPS = _NP // _NS
_AT = 10000
_DRW = 80
_NB = _NP // _D
_L = 16


def _tc1_body(x_ref, w_ref, as_ref, ad_ref, h_ref, asrc_ref, adst_ref,
              ss_ref, c_ref):
    h = lax.dot_general(x_ref[...], w_ref[...], (((1,), (0,)), ((), ())),
                        preferred_element_type=jnp.float32)
    h_ref[0:_N, :] = h
    h_ref[_N:_NP, :] = jnp.zeros((_NP - _N, _D), jnp.float32)
    a_s = lax.dot_general(h, as_ref[...], (((1,), (0,)), ((), ())),
                          preferred_element_type=jnp.float32)
    a_d = lax.dot_general(h, ad_ref[...], (((1,), (0,)), ((), ())),
                          preferred_element_type=jnp.float32)
    asrc_ref[0:_N, :] = a_s
    asrc_ref[_N:_NP, :] = jnp.zeros((_NP - _N, 1), jnp.float32)
    adst_ref[0:_N, :] = a_d
    adst_ref[_N:_NP, :] = jnp.zeros((_NP - _N, 1), jnp.float32)
    t = jnp.max(a_s) + jnp.max(a_d)
    c = jnp.maximum(t, 0.2 * t)
    u = a_s + a_d
    ss_ref[0:_N, :] = jnp.exp(jnp.maximum(u, 0.2 * u) - c)
    ss_ref[_N:_NP, :] = jnp.ones((_NP - _N, 1), jnp.float32)
    c_ref[...] = jnp.broadcast_to(c, (1, 1))


_tc1 = pl.pallas_call(
    _tc1_body,
    out_shape=[
        jax.ShapeDtypeStruct((_NP, _D), jnp.float32),
        jax.ShapeDtypeStruct((_NP, 1), jnp.float32),
        jax.ShapeDtypeStruct((_NP, 1), jnp.float32),
        jax.ShapeDtypeStruct((_NP, 1), jnp.float32),
        jax.ShapeDtypeStruct((1, 1), jnp.float32),
    ],
)


def _sc1_body(h_hbm, src_hbm, dst_hbm, asrc_hbm, adst_hbm, cvec_hbm, zrows_hbm,
              parts_hbm, denp_hbm,
              asrc_v, adst_v, c_v,
              srci0, srci1, srci2, srci3, dsti0, dsti1, dsti2, dsti3,
              sv0, sv1, sv2, sv3, rows0, rows1, rows2, den_v, idn_v,
              acc_sh, den_sh,
              semi0, semi1, semi2, semi3, semg0, semg1, semg2,
              sems0, sems1, sems2):
    cid = lax.axis_index("c")
    sid = lax.axis_index("s")
    wid = cid * _NS + sid

    pltpu.sync_copy(zrows_hbm, acc_sh.at[pl.ds(sid * _RPS, _RPS)])
    pltpu.sync_copy(zrows_hbm.at[pl.ds(0, _DRW // _NS)],
                    den_sh.at[pl.ds(sid * (_DRW // _NS), _DRW // _NS)])
    pltpu.sync_copy(zrows_hbm.at[pl.ds(0, _DRW)], den_v)
    pltpu.sync_copy(asrc_hbm.at[pl.ds(0, _AT)], asrc_v)
    pltpu.sync_copy(adst_hbm.at[pl.ds(0, _AT)], adst_v)
    pltpu.sync_copy(cvec_hbm, c_v)
    for g in range(_DRW // _L):
        idn_v[0, pl.ds(g * _L, _L)] = g * _L + lax.iota(jnp.int32, _L)
    plsc.subcore_barrier()

    creg = c_v[...]
    basew = wid * _EPW
    ibufs = ((srci0, dsti0, sv0, semi0), (srci1, dsti1, sv1, semi1),
             (srci2, dsti2, sv2, semi2), (srci3, dsti3, sv3, semi3))
    rbufs = ((rows0, semg0, sems0), (rows1, semg1, sems1),
             (rows2, semg2, sems2))

    def idx_start(base, ib):
        srci, dsti, _, semi = ib
        pltpu.async_copy(src_hbm.at[pl.ds(base, _CH)], srci, semi)
        pltpu.async_copy(dst_hbm.at[pl.ds(base, _CH)], dsti.at[0], semi)

    def idx_wait(base, ib):
        srci, dsti, _, semi = ib
        pltpu.make_async_copy(src_hbm.at[pl.ds(base, _CH)], srci, semi).wait()
        pltpu.make_async_copy(dst_hbm.at[pl.ds(base, _CH)], dsti.at[0],
                              semi).wait()

    def gather_start(ib, rb):
        pltpu.async_copy(h_hbm.at[ib[0]], rb[0], rb[1])

    def gather_wait(ib, rb):
        pltpu.make_async_copy(h_hbm.at[ib[0]], rb[0], rb[1]).wait()

    def scatter_start(ib, rb):
        pltpu.async_copy(rb[0], acc_sh.at[ib[1].at[0]], rb[2], add=True)

    def scatter_wait(ib, rb):
        pltpu.make_async_copy(rb[0], acc_sh.at[ib[1].at[0]], rb[2]).wait()

    def weights(ib):
        srci, dsti, s_v, _ = ib
        for g in range(_CH // _L):
            sv = srci[pl.ds(g * _L, _L)]
            dv = dsti[0, pl.ds(g * _L, _L)]
            u = plsc.load_gather(asrc_v, [sv]) + plsc.load_gather(adst_v, [dv])
            e = jnp.maximum(u, 0.2 * u)
            s = jnp.exp(e - creg)
            s_v[pl.ds(g * _L, _L)] = s
            plsc.addupdate_scatter(den_v, [dv >> 7, dv & 127], s)

    def scale(ib, rb):
        s_v = ib[2]
        rows = rb[0]

        @plsc.parallel_loop(0, _CH, unroll=4)
        def _(r):
            sp = plsc.load_gather(s_v, [jnp.full((_L,), r, jnp.int32)])
            for j in range(_D // _L):
                sl = (r, pl.ds(j * _L, _L))
                rows[sl] = rows[sl] * sp

    idx_start(basew, ibufs[0])
    idx_wait(basew, ibufs[0])
    gather_start(ibufs[0], rbufs[0])
    idx_start(basew + _CH, ibufs[1])
    weights(ibufs[0])

    @pl.loop(0, _NCHM // 12)
    def _(i):
        for b in range(12):
            k = i * 12 + b
            ib_cur, rb_cur = ibufs[b % 4], rbufs[b % 3]
            ib_nxt, rb_nxt = ibufs[(b + 1) % 4], rbufs[(b + 1) % 3]
            ib_pre = ibufs[(b + 2) % 4]

            if b >= 2:
                scatter_wait(ib_pre, rb_nxt)
            else:
                @pl.when(k >= 2)
                def _():
                    scatter_wait(ib_pre, rb_nxt)

            def prep_next():
                idx_wait(basew + (k + 1) * _CH, ib_nxt)
                gather_start(ib_nxt, rb_nxt)

            def prefetch_idx():
                idx_start(basew + (k + 2) * _CH, ib_pre)

            if b == 11:
                @pl.when(k + 1 < _NCHM)
                def _():
                    prep_next()
            else:
                prep_next()
            if b >= 10:
                @pl.when(k + 2 < _NCHM)
                def _():
                    prefetch_idx()
            else:
                prefetch_idx()

            gather_wait(ib_cur, rb_cur)
            scale(ib_cur, rb_cur)
            scatter_start(ib_cur, rb_cur)

            if b == 11:
                @pl.when(k + 1 < _NCHM)
                def _():
                    weights(ib_nxt)
            else:
                weights(ib_nxt)

    scatter_wait(ibufs[310 % 4], rbufs[310 % 3])
    scatter_wait(ibufs[311 % 4], rbufs[311 % 3])

    @pl.when(wid < 16)
    def _():
        base = _REM_BASE + wid * _CH
        idx_start(base, ibufs[0])
        idx_wait(base, ibufs[0])
        gather_start(ibufs[0], rbufs[0])
        weights(ibufs[0])
        gather_wait(ibufs[0], rbufs[0])
        scale(ibufs[0], rbufs[0])
        pltpu.sync_copy(rbufs[0][0], acc_sh.at[ibufs[0][1].at[0]], add=True)

    pltpu.sync_copy(den_v, den_sh.at[idn_v.at[0]], add=True)
    plsc.subcore_barrier()

    pltpu.sync_copy(acc_sh.at[pl.ds(sid * _RPS, _RPS)],
                    parts_hbm.at[cid].at[pl.ds(sid * _RPS, _RPS)])

    @pl.when(sid == 0)
    def _():
        pltpu.sync_copy(den_sh, denp_hbm.at[cid])


_sc_params = pltpu.CompilerParams()
if "needs_layout_passes" in pltpu.CompilerParams.__dataclass_fields__:
    _sc_params = dataclasses.replace(_sc_params, needs_layout_passes=False)

_sc1 = pl.kernel(
    _sc1_body,
    out_type=[
        jax.ShapeDtypeStruct((_NC, _NP, _D), jnp.float32),
        jax.ShapeDtypeStruct((_NC, _DRW, _D), jnp.float32),
    ],
    mesh=plsc.VectorSubcoreMesh(core_axis_name="c", subcore_axis_name="s"),
    compiler_params=_sc_params,
    scratch_types=[
        pltpu.VMEM((_AT,), jnp.float32),
        pltpu.VMEM((_AT,), jnp.float32),
        pltpu.VMEM((_L,), jnp.float32),
        pltpu.VMEM((_CH,), jnp.int32),
        pltpu.VMEM((_CH,), jnp.int32),
        pltpu.VMEM((_CH,), jnp.int32),
        pltpu.VMEM((_CH,), jnp.int32),
        pltpu.VMEM((1, _CH), jnp.int32),
        pltpu.VMEM((1, _CH), jnp.int32),
        pltpu.VMEM((1, _CH), jnp.int32),
        pltpu.VMEM((1, _CH), jnp.int32),
        pltpu.VMEM((_CH,), jnp.float32),
        pltpu.VMEM((_CH,), jnp.float32),
        pltpu.VMEM((_CH,), jnp.float32),
        pltpu.VMEM((_CH,), jnp.float32),
        pltpu.VMEM((_CH, _D), jnp.float32),
        pltpu.VMEM((_CH, _D), jnp.float32),
        pltpu.VMEM((_CH, _D), jnp.float32),
        pltpu.VMEM((_DRW, _D), jnp.float32),
        pltpu.VMEM((1, _DRW), jnp.int32),
        pltpu.VMEM_SHARED((_NP, _D), jnp.float32),
        pltpu.VMEM_SHARED((_DRW, _D), jnp.float32),
        pltpu.SemaphoreType.DMA,
        pltpu.SemaphoreType.DMA,
        pltpu.SemaphoreType.DMA,
        pltpu.SemaphoreType.DMA,
        pltpu.SemaphoreType.DMA,
        pltpu.SemaphoreType.DMA,
        pltpu.SemaphoreType.DMA,
        pltpu.SemaphoreType.DMA,
        pltpu.SemaphoreType.DMA,
        pltpu.SemaphoreType.DMA,
    ],
)


def _sc2_body(parts_hbm, denp_hbm, h_hbm, ss_hbm, outd_hbm,
              p0_v, p1_v, h_v, ss_v, d0_v, d1_v, sem):
    cid = lax.axis_index("c")
    sid = lax.axis_index("s")
    wid = cid * _NS + sid

    pltpu.sync_copy(denp_hbm.at[0], d0_v)
    pltpu.sync_copy(denp_hbm.at[1], d1_v)

    nblk = jnp.where(wid < 15, 3, 2)
    blk0 = jnp.where(wid < 15, 3 * wid, 2 * wid + 15)

    for q in range(3):
        @pl.when(q < nblk)
        def _():
            j = blk0 + q
            base = j * _D
            pltpu.sync_copy(parts_hbm.at[0].at[pl.ds(base, _D)], p0_v)
            pltpu.sync_copy(parts_hbm.at[1].at[pl.ds(base, _D)], p1_v)
            pltpu.sync_copy(h_hbm.at[pl.ds(base, _D)], h_v)
            pltpu.sync_copy(ss_hbm.at[pl.ds(base, _D)], ss_v)

            @plsc.parallel_loop(0, _D, unroll=4)
            def _(r):
                jrow = jnp.full((_L,), j, jnp.int32)
                rcol = jnp.full((_L,), r, jnp.int32)
                den = (plsc.load_gather(d0_v, [jrow, rcol]) +
                       plsc.load_gather(d1_v, [jrow, rcol]))
                ssr = plsc.load_gather(ss_v, [rcol])
                rec = 1.0 / (den + ssr + 1e-16)
                for jj in range(_D // _L):
                    sl = (r, pl.ds(jj * _L, _L))
                    p0_v[sl] = (p0_v[sl] + p1_v[sl] + ssr * h_v[sl]) * rec

            pltpu.sync_copy(p0_v, outd_hbm.at[pl.ds(base, _D)])


_sc2 = pl.kernel(
    _sc2_body,
    out_type=jax.ShapeDtypeStruct((_NP, _D), jnp.float32),
    mesh=plsc.VectorSubcoreMesh(core_axis_name="c", subcore_axis_name="s"),
    compiler_params=_sc_params,
    scratch_types=[
        pltpu.VMEM((_D, _D), jnp.float32),
        pltpu.VMEM((_D, _D), jnp.float32),
        pltpu.VMEM((_D, _D), jnp.float32),
        pltpu.VMEM((_D,), jnp.float32),
        pltpu.VMEM((_DRW, _D), jnp.float32),
        pltpu.VMEM((_DRW, _D), jnp.float32),
        pltpu.SemaphoreType.DMA,
    ],
)


def _tc2_body(outd_ref, b_ref, o_ref):
    o_ref[...] = jnp.tanh(outd_ref[0:_N, :] + b_ref[...])


_tc2 = pl.pallas_call(
    _tc2_body,
    out_shape=jax.ShapeDtypeStruct((_N, _D), jnp.float32),
)


@jax.jit
def kernel(x, edge_index, W, att_src, att_dst, bias):
    src = edge_index[0]
    dst = edge_index[1]
    hp, asrc, adst, ssp, c = _tc1(x, W, att_src.reshape(_D, 1),
                                  att_dst.reshape(_D, 1))
    cvec = jnp.broadcast_to(c.reshape(1), (_L,))
    zrows = jnp.zeros((_RPS, _D), jnp.float32)
    parts, denp = _sc1(hp, src, dst, asrc.reshape(_NP), adst.reshape(_NP),
                       cvec, zrows)
    outd = _sc2(parts, denp, hp, ssp.reshape(_NP))
    return _tc2(outd, bias.reshape(1, _D))
